# baseline (speedup 1.0000x reference)
import jax
import jax.numpy as jnp
from jax.experimental import pallas as pl
from jax.experimental.pallas import tpu as pltpu

_HIDDEN = 30
_LANE = 128
_BLOCK_B = 16384


def _anet_fused_kernel(x_ref, w1_ref, b1_ref, w2_ref, b2_ref, o_ref):
    h = jnp.dot(x_ref[...], w1_ref[...], preferred_element_type=jnp.float32)
    h = jnp.maximum(h + b1_ref[...], 0.0)
    y = jnp.dot(h, w2_ref[...], preferred_element_type=jnp.float32)
    y = jnp.tanh(y + b2_ref[...]) * 2.0
    rows, lanes = o_ref.shape
    pack = lanes // y.shape[1]
    if pack == 1:
        o_ref[...] = y
    else:
        y3 = y.reshape(rows, pack, y.shape[1])
        o_ref[...] = jnp.concatenate(
            [y3[:, k, :] for k in range(pack)], axis=1)


def kernel(x, w1, b1, w2, b2):
    B, s_dim = x.shape
    a_dim = w2.shape[1]
    x = x.astype(jnp.float32)

    h_pad = max(_LANE, ((_HIDDEN + _LANE - 1) // _LANE) * _LANE)
    w1p = jnp.zeros((s_dim, h_pad), jnp.float32).at[:, :_HIDDEN].set(
        w1.astype(jnp.float32))
    b1p = jnp.zeros((1, h_pad), jnp.float32).at[:, :_HIDDEN].set(
        jnp.reshape(b1, (1, -1)).astype(jnp.float32))
    w2p = jnp.zeros((h_pad, a_dim), jnp.float32).at[:_HIDDEN, :].set(
        w2.astype(jnp.float32))
    b2p = jnp.reshape(b2, (1, a_dim)).astype(jnp.float32)

    pack = _LANE // a_dim if (a_dim <= _LANE and _LANE % a_dim == 0) else 1

    block_b = min(_BLOCK_B, B)
    pad_b = (-B) % (block_b if B > block_b else max(pack, 8))
    if pad_b:
        x = jnp.pad(x, ((0, pad_b), (0, 0)))
    bp = B + pad_b
    block_b = min(block_b, bp)
    nb = bp // block_b

    out = pl.pallas_call(
        _anet_fused_kernel,
        out_shape=jax.ShapeDtypeStruct((bp // pack, pack * a_dim),
                                       jnp.float32),
        grid=(nb,),
        in_specs=[
            pl.BlockSpec((block_b, s_dim), lambda i: (i, 0)),
            pl.BlockSpec((s_dim, h_pad), lambda i: (0, 0)),
            pl.BlockSpec((1, h_pad), lambda i: (0, 0)),
            pl.BlockSpec((h_pad, a_dim), lambda i: (0, 0)),
            pl.BlockSpec((1, a_dim), lambda i: (0, 0)),
        ],
        out_specs=pl.BlockSpec((block_b // pack, pack * a_dim),
                               lambda i: (i, 0)),
        compiler_params=pltpu.CompilerParams(
            dimension_semantics=("parallel",)),
    )(x, w1p, b1p, w2p, b2p)

    return out.reshape(bp, a_dim)[:B]


# speedup vs baseline: 1.5652x; 1.5652x over previous
import jax
import jax.numpy as jnp
from jax.experimental import pallas as pl
from jax.experimental.pallas import tpu as pltpu

_BLOCK_B = 16384


def _anet_fused_kernel(x_ref, w1_ref, b1_ref, w2_ref, b2_ref, o_ref):
    h = jnp.dot(x_ref[...], w1_ref[...], preferred_element_type=jnp.float32)
    h = jnp.maximum(h + b1_ref[...], 0.0)
    y = jnp.dot(h, w2_ref[...], preferred_element_type=jnp.float32)
    o_ref[...] = jnp.tanh(y + b2_ref[...]) * 2.0


def kernel(x, w1, b1, w2, b2):
    B, s_dim = x.shape
    hidden = w1.shape[1]
    a_dim = w2.shape[1]
    x = x.astype(jnp.float32)
    w1 = w1.astype(jnp.float32)
    w2 = w2.astype(jnp.float32)
    b1 = jnp.reshape(b1, (1, hidden)).astype(jnp.float32)
    b2 = jnp.reshape(b2, (1, a_dim)).astype(jnp.float32)

    block_b = min(_BLOCK_B, B)
    pad_b = (-B) % (block_b if B > block_b else 8)
    if pad_b:
        x = jnp.pad(x, ((0, pad_b), (0, 0)))
    bp = B + pad_b
    block_b = min(block_b, bp)
    nb = bp // block_b

    out = pl.pallas_call(
        _anet_fused_kernel,
        out_shape=jax.ShapeDtypeStruct((bp, a_dim), jnp.float32),
        grid=(nb,),
        in_specs=[
            pl.BlockSpec((block_b, s_dim), lambda i: (i, 0)),
            pl.BlockSpec((s_dim, hidden), lambda i: (0, 0)),
            pl.BlockSpec((1, hidden), lambda i: (0, 0)),
            pl.BlockSpec((hidden, a_dim), lambda i: (0, 0)),
            pl.BlockSpec((1, a_dim), lambda i: (0, 0)),
        ],
        out_specs=pl.BlockSpec((block_b, a_dim), lambda i: (i, 0)),
        compiler_params=pltpu.CompilerParams(
            dimension_semantics=("arbitrary",)),
    )(x, w1, b1, w2, b2)

    return out[:B]


# speedup vs baseline: 4.0887x; 2.6123x over previous
import jax
import jax.numpy as jnp
from jax.experimental import pallas as pl
from jax.experimental.pallas import tpu as pltpu

_BLOCK_B = 16384


def _anet_fused_kernel(x_ref, w1_ref, b1_ref, w2_ref, b2_ref, o_ref):
    h = jnp.dot(x_ref[...], w1_ref[...], preferred_element_type=jnp.float32)
    h = jnp.maximum(h + b1_ref[...], 0.0)
    y = jnp.dot(h, w2_ref[...], preferred_element_type=jnp.float32)
    y = jnp.tanh(y + b2_ref[...]) * 2.0
    o_ref[...] = y.T


def kernel(x, w1, b1, w2, b2):
    B, s_dim = x.shape
    hidden = w1.shape[1]
    a_dim = w2.shape[1]
    x = x.astype(jnp.float32)
    w1 = w1.astype(jnp.float32)
    w2 = w2.astype(jnp.float32)
    b1 = jnp.reshape(b1, (1, hidden)).astype(jnp.float32)
    b2 = jnp.reshape(b2, (1, a_dim)).astype(jnp.float32)

    block_b = min(_BLOCK_B, B)
    pad_b = (-B) % (block_b if B > block_b else 8)
    if pad_b:
        x = jnp.pad(x, ((0, pad_b), (0, 0)))
    bp = B + pad_b
    block_b = min(block_b, bp)
    nb = bp // block_b

    out_t = pl.pallas_call(
        _anet_fused_kernel,
        out_shape=jax.ShapeDtypeStruct((a_dim, bp), jnp.float32),
        grid=(nb,),
        in_specs=[
            pl.BlockSpec((block_b, s_dim), lambda i: (i, 0)),
            pl.BlockSpec((s_dim, hidden), lambda i: (0, 0)),
            pl.BlockSpec((1, hidden), lambda i: (0, 0)),
            pl.BlockSpec((hidden, a_dim), lambda i: (0, 0)),
            pl.BlockSpec((1, a_dim), lambda i: (0, 0)),
        ],
        out_specs=pl.BlockSpec((a_dim, block_b), lambda i: (0, i)),
        compiler_params=pltpu.CompilerParams(
            dimension_semantics=("arbitrary",)),
    )(x, w1, b1, w2, b2)

    return out_t[:, :B].T


# speedup vs baseline: 4.1444x; 1.0136x over previous
import jax
import jax.numpy as jnp
from jax.experimental import pallas as pl
from jax.experimental.pallas import tpu as pltpu

_BLOCK_B = 32768


def _anet_fused_kernel(x_ref, w1_ref, b1_ref, w2_ref, b2_ref, o_ref):
    h = jnp.dot(x_ref[...], w1_ref[...], preferred_element_type=jnp.float32)
    h = jnp.maximum(h + b1_ref[...], 0.0)
    y = jnp.dot(h, w2_ref[...], preferred_element_type=jnp.float32)
    y = jnp.tanh(y + b2_ref[...]) * 2.0
    o_ref[...] = y.T


def kernel(x, w1, b1, w2, b2):
    B, s_dim = x.shape
    hidden = w1.shape[1]
    a_dim = w2.shape[1]
    x = x.astype(jnp.float32)
    w1 = w1.astype(jnp.float32)
    w2 = w2.astype(jnp.float32)
    b1 = jnp.reshape(b1, (1, hidden)).astype(jnp.float32)
    b2 = jnp.reshape(b2, (1, a_dim)).astype(jnp.float32)

    block_b = min(_BLOCK_B, B)
    pad_b = (-B) % (block_b if B > block_b else 8)
    if pad_b:
        x = jnp.pad(x, ((0, pad_b), (0, 0)))
    bp = B + pad_b
    block_b = min(block_b, bp)
    nb = bp // block_b

    out_t = pl.pallas_call(
        _anet_fused_kernel,
        out_shape=jax.ShapeDtypeStruct((a_dim, bp), jnp.float32),
        grid=(nb,),
        in_specs=[
            pl.BlockSpec((block_b, s_dim), lambda i: (i, 0)),
            pl.BlockSpec((s_dim, hidden), lambda i: (0, 0)),
            pl.BlockSpec((1, hidden), lambda i: (0, 0)),
            pl.BlockSpec((hidden, a_dim), lambda i: (0, 0)),
            pl.BlockSpec((1, a_dim), lambda i: (0, 0)),
        ],
        out_specs=pl.BlockSpec((a_dim, block_b), lambda i: (0, i)),
        compiler_params=pltpu.CompilerParams(
            dimension_semantics=("arbitrary",)),
    )(x, w1, b1, w2, b2)

    return out_t[:, :B].T


# speedup vs baseline: 4.5631x; 1.1010x over previous
import jax
import jax.numpy as jnp
from jax.experimental import pallas as pl
from jax.experimental.pallas import tpu as pltpu

_BLOCK_B = 32768


def _anet_fused_kernel(x_ref, w1t_ref, b1_ref, w2t_ref, b2_ref, o_ref):
    h = jax.lax.dot_general(
        x_ref[...], w1t_ref[...], (((1,), (1,)), ((), ())),
        preferred_element_type=jnp.float32)
    h = jnp.maximum(h + b1_ref[...], 0.0)
    y = jax.lax.dot_general(
        h, w2t_ref[...], (((1,), (1,)), ((), ())),
        preferred_element_type=jnp.float32)
    y = jnp.tanh(y + b2_ref[...]) * 2.0
    o_ref[...] = y.T


def kernel(x, w1, b1, w2, b2):
    B, s_dim = x.shape
    hidden = w1.shape[1]
    a_dim = w2.shape[1]
    x = x.astype(jnp.float32)
    w1t = jnp.transpose(w1).astype(jnp.float32)
    w2t = jnp.transpose(w2).astype(jnp.float32)
    b1 = jnp.reshape(b1, (1, hidden)).astype(jnp.float32)
    b2 = jnp.reshape(b2, (1, a_dim)).astype(jnp.float32)

    block_b = min(_BLOCK_B, B)
    pad_b = (-B) % (block_b if B > block_b else 8)
    if pad_b:
        x = jnp.pad(x, ((0, pad_b), (0, 0)))
    bp = B + pad_b
    block_b = min(block_b, bp)
    nb = bp // block_b

    out_t = pl.pallas_call(
        _anet_fused_kernel,
        out_shape=jax.ShapeDtypeStruct((a_dim, bp), jnp.float32),
        grid=(nb,),
        in_specs=[
            pl.BlockSpec((block_b, s_dim), lambda i: (i, 0)),
            pl.BlockSpec((hidden, s_dim), lambda i: (0, 0)),
            pl.BlockSpec((1, hidden), lambda i: (0, 0)),
            pl.BlockSpec((a_dim, hidden), lambda i: (0, 0)),
            pl.BlockSpec((1, a_dim), lambda i: (0, 0)),
        ],
        out_specs=pl.BlockSpec((a_dim, block_b), lambda i: (0, i)),
        compiler_params=pltpu.CompilerParams(
            dimension_semantics=("arbitrary",)),
    )(x, w1t, b1, w2t, b2)

    return out_t[:, :B].T
